# Initial kernel scaffold; baseline (speedup 1.0000x reference)
import jax
import jax.numpy as jnp
from jax.experimental import pallas as pl


def kernel(node_features, edge_index, angles, gt_edges, nc1_W1, nc1_b1, nc1_W2, nc1_b2, nc2_W1, nc2_b1, nc2_W2, nc2_b2, ec1_W1, ec1_b1, ec1_W2, ec1_b2, ec1_W3, ec1_b3, ec2_W1, ec2_b1, ec2_W2, ec2_b2, ec2_W3, ec2_b3):
    raise NotImplementedError("write your pallas kernel here")


# speedup vs baseline: 2.4002x; 2.4002x over previous
import functools

import jax
import jax.numpy as jnp
from jax import lax
from jax.experimental import pallas as pl
from jax.experimental.pallas import tpu as pltpu
from jax.experimental.pallas import tpu_sc as plsc

N = 10000
E = 320000
D = 128
OUT = 32

NC = 2
NS = 16
NW = NC * NS
EPW = E // NW
K = 80
NCHUNK = EPW // K
SROW = D + 16
NP = 10240
ZR = 80
RPT = NP // NS

_MESH = plsc.VectorSubcoreMesh(
    core_axis_name="c", subcore_axis_name="s", num_cores=NC, num_subcores=NS)


def _make_sc_scatter(width):
    ngrp = D // 16
    wgrp = width // 16

    @functools.partial(
        pl.kernel,
        out_type=jax.ShapeDtypeStruct((NC, NP, width), jnp.float32),
        mesh=_MESH,
        compiler_params=pltpu.CompilerParams(use_tc_tiling_on_sc=False),
        scratch_types=[
            pltpu.VMEM((K,), jnp.int32),
            pltpu.VMEM((K,), jnp.int32),
            pltpu.VMEM((K, D), jnp.float32),
            pltpu.VMEM((K, D), jnp.float32),
            pltpu.VMEM((K, width), jnp.float32),
            pltpu.VMEM_SHARED((NP, width), jnp.float32),
            pltpu.SemaphoreType.DMA,
            pltpu.SemaphoreType.DMA,
        ],
    )
    def kern(qa, qb, src, dst, out, idx_s, idx_d, abuf, bbuf, rbuf,
             s_sh, sem_a, sem_b):
        cid = lax.axis_index("c")
        sid = lax.axis_index("s")
        wid = sid * NC + cid

        zero16 = jnp.zeros((16,), jnp.float32)

        def zrow(r, carry):
            for g in range(wgrp):
                rbuf[r, pl.ds(g * 16, 16)] = zero16
            return carry
        lax.fori_loop(0, ZR, zrow, 0)
        for j in range(RPT // ZR):
            row0 = sid * RPT + j * ZR
            pltpu.sync_copy(rbuf, s_sh.at[pl.ds(row0, ZR)])

        if width > D:
            lane = lax.broadcasted_iota(jnp.int32, (16,), 0)
            onehot = jnp.where(lane == 0, 1.0, 0.0).astype(jnp.float32)
            def irow(r, carry):
                rbuf[r, pl.ds(D, 16)] = onehot
                return carry
            lax.fori_loop(0, K, irow, 0)

        plsc.subcore_barrier()

        def chunk(i, carry):
            base = wid * EPW + i * K
            pltpu.sync_copy(src.at[pl.ds(base, K)], idx_s)
            pltpu.sync_copy(dst.at[pl.ds(base, K)], idx_d)
            cp_a = pltpu.async_copy(qa.at[idx_s], abuf, sem_a)
            cp_b = pltpu.async_copy(qb.at[idx_d], bbuf, sem_b)
            cp_a.wait()
            cp_b.wait()

            def row(r, c2):
                for g in range(ngrp):
                    sl = pl.ds(g * 16, 16)
                    rbuf[r, sl] = jnp.maximum(abuf[r, sl] + bbuf[r, sl], 0.0)
                return c2
            lax.fori_loop(0, K, row, 0)
            pltpu.sync_copy(rbuf, s_sh.at[idx_d], add=True)
            return carry
        lax.fori_loop(0, NCHUNK, chunk, 0)

        plsc.subcore_barrier()

        for j in range(RPT // ZR):
            row0 = sid * RPT + j * ZR
            pltpu.sync_copy(s_sh.at[pl.ds(row0, ZR)], rbuf)
            pltpu.sync_copy(rbuf, out.at[cid, pl.ds(row0, ZR)])

    return kern


_sc_scatter_deg = _make_sc_scatter(SROW)
_sc_scatter_plain = _make_sc_scatter(D)


@functools.partial(
    pl.kernel,
    out_type=[jax.ShapeDtypeStruct((E, D), jnp.float32),
              jax.ShapeDtypeStruct((E, D), jnp.float32)],
    mesh=_MESH,
    scratch_types=[
        pltpu.VMEM((K,), jnp.int32),
        pltpu.VMEM((K,), jnp.int32),
        pltpu.VMEM((K, D), jnp.float32),
        pltpu.VMEM((K, D), jnp.float32),
        pltpu.VMEM((K, D), jnp.float32),
        pltpu.VMEM((K, D), jnp.float32),
        pltpu.VMEM((K, D), jnp.float32),
        pltpu.VMEM((K, D), jnp.float32),
        pltpu.SemaphoreType.DMA,
    ],
)
def _sc_edge(pa, pb, src, dst, r_out, d_out, idx_s, idx_d,
             abuf, bbuf, cbuf, dgbuf, rbuf, dbuf, sem):
    cid = lax.axis_index("c")
    sid = lax.axis_index("s")
    wid = sid * NC + cid
    ngrp = D // 16

    def chunk(i, carry):
        base = wid * EPW + i * K
        pltpu.sync_copy(src.at[pl.ds(base, K)], idx_s)
        pltpu.sync_copy(dst.at[pl.ds(base, K)], idx_d)
        cp1 = pltpu.async_copy(pa.at[idx_s], abuf, sem)
        cp2 = pltpu.async_copy(pb.at[idx_d], bbuf, sem)
        cp3 = pltpu.async_copy(pa.at[idx_d], cbuf, sem)
        cp4 = pltpu.async_copy(pb.at[idx_s], dgbuf, sem)
        cp1.wait()
        cp2.wait()
        cp3.wait()
        cp4.wait()

        def row(r, c2):
            for g in range(ngrp):
                sl = pl.ds(g * 16, 16)
                ru = jnp.maximum(abuf[r, sl] + bbuf[r, sl], 0.0)
                rv = jnp.maximum(cbuf[r, sl] + dgbuf[r, sl], 0.0)
                rbuf[r, sl] = (ru + rv) * 0.5
                dbuf[r, sl] = ru - rv
            return c2
        lax.fori_loop(0, K, row, 0)
        pltpu.sync_copy(rbuf, r_out.at[pl.ds(base, K)])
        pltpu.sync_copy(dbuf, d_out.at[pl.ds(base, K)])
        return carry
    lax.fori_loop(0, NCHUNK, chunk, 0)


BN = 2000
BE = 3200


def _dot(a, b):
    return jax.lax.dot_general(a, b, (((1,), (0,)), ((), ())),
                               preferred_element_type=jnp.float32)


def _prep_body(x_ref, w_ref, b_ref, qa_ref, qb_ref):
    x = x_ref[...]
    qa_ref[...] = _dot(x, w_ref[:D])
    qb_ref[...] = _dot(x, w_ref[D:]) + b_ref[...]


def _prep(x, w1, b1):
    return pl.pallas_call(
        _prep_body,
        grid=(N // BN,),
        in_specs=[
            pl.BlockSpec((BN, D), lambda i: (i, 0)),
            pl.BlockSpec((2 * D, D), lambda i: (0, 0)),
            pl.BlockSpec((1, D), lambda i: (0, 0)),
        ],
        out_specs=[pl.BlockSpec((BN, D), lambda i: (i, 0))] * 2,
        out_shape=[jax.ShapeDtypeStruct((N, D), jnp.float32)] * 2,
    )(x, w1, b1)


def _mid1_body(sp_ref, w2_ref, b2_ref, ew1_ref, eb1_ref, nw1_ref, nb1_ref,
               x1_ref, p1a_ref, p1b_ref, q2a_ref, q2b_ref):
    s = sp_ref[0] + sp_ref[1]
    deg = s[:, D:D + 1]
    agg = _dot(s[:, :D], w2_ref[...]) + deg * b2_ref[...]
    x1 = jnp.maximum(agg / jnp.maximum(deg, 1.0), 0.0)
    x1_ref[...] = x1
    p1a_ref[...] = _dot(x1, ew1_ref[:D])
    p1b_ref[...] = _dot(x1, ew1_ref[D:]) + eb1_ref[...]
    mu = jnp.mean(x1, axis=1, keepdims=True)
    var = jnp.mean((x1 - mu) ** 2, axis=1, keepdims=True)
    xn = (x1 - mu) / (jnp.sqrt(var) + 1e-6)
    q2a_ref[...] = _dot(xn, nw1_ref[:D])
    q2b_ref[...] = _dot(xn, nw1_ref[D:]) + nb1_ref[...]


def _mid1(s1, w2, b2, ew1, eb1, nw1, nb1):
    return pl.pallas_call(
        _mid1_body,
        grid=(N // BN,),
        in_specs=[
            pl.BlockSpec((NC, BN, SROW), lambda i: (0, i, 0)),
            pl.BlockSpec((D, D), lambda i: (0, 0)),
            pl.BlockSpec((1, D), lambda i: (0, 0)),
            pl.BlockSpec((2 * D, D), lambda i: (0, 0)),
            pl.BlockSpec((1, D), lambda i: (0, 0)),
            pl.BlockSpec((2 * D, D), lambda i: (0, 0)),
            pl.BlockSpec((1, D), lambda i: (0, 0)),
        ],
        out_specs=[pl.BlockSpec((BN, D), lambda i: (i, 0))] * 5,
        out_shape=[jax.ShapeDtypeStruct((N, D), jnp.float32)] * 5,
    )(s1, w2, b2, ew1, eb1, nw1, nb1)


def _mid2_body(s1p_ref, s2p_ref, x1_ref, w2_ref, b2_ref, ew1_ref, eb1_ref,
               p2a_ref, p2b_ref):
    deg = s1p_ref[0, :, D:D + 1] + s1p_ref[1, :, D:D + 1]
    s2 = s2p_ref[0] + s2p_ref[1]
    agg = _dot(s2, w2_ref[...]) + deg * b2_ref[...]
    x2 = x1_ref[...] + jnp.maximum(agg / jnp.maximum(deg, 1.0), 0.0)
    p2a_ref[...] = _dot(x2, ew1_ref[:D])
    p2b_ref[...] = _dot(x2, ew1_ref[D:]) + eb1_ref[...]


def _mid2(s1, s2, x1, w2, b2, ew1, eb1):
    return pl.pallas_call(
        _mid2_body,
        grid=(N // BN,),
        in_specs=[
            pl.BlockSpec((NC, BN, SROW), lambda i: (0, i, 0)),
            pl.BlockSpec((NC, BN, D), lambda i: (0, i, 0)),
            pl.BlockSpec((BN, D), lambda i: (i, 0)),
            pl.BlockSpec((D, D), lambda i: (0, 0)),
            pl.BlockSpec((1, D), lambda i: (0, 0)),
            pl.BlockSpec((2 * D, D), lambda i: (0, 0)),
            pl.BlockSpec((1, D), lambda i: (0, 0)),
        ],
        out_specs=[pl.BlockSpec((BN, D), lambda i: (i, 0))] * 2,
        out_shape=[jax.ShapeDtypeStruct((N, D), jnp.float32)] * 2,
    )(s1, s2, x1, w2, b2, ew1, eb1)


def _final_body(r1_ref, d1_ref, r2_ref, d2_ref, ang_ref,
                e1w2_ref, e1w3_ref, e1b2_ref, e1b3_ref,
                e2w2_ref, e2w3_ref, e2b2_ref, e2b3_ref,
                e2_ref, ssq_ref):
    i = pl.program_id(0)
    w3a1 = e1w3_ref[:D]
    v1 = e1w3_ref[D:D + 1]
    m1 = _dot(e1w2_ref[...], w3a1)
    c1 = _dot(e1b2_ref[...], w3a1) + e1b3_ref[...]
    w3a2 = e2w3_ref[:D]
    w3b2 = e2w3_ref[D:]
    a2 = _dot(e2w2_ref[...], w3a2)
    a1 = _dot(m1, w3b2)
    wv = _dot(v1, w3b2)
    cc = _dot(c1, w3b2) + _dot(e2b2_ref[...], w3a2) + e2b3_ref[...]

    e2 = (_dot(r2_ref[...], a2) + _dot(r1_ref[...], a1)
          + ang_ref[...] * wv + cc)
    e2_ref[...] = e2

    t1 = _dot(d1_ref[...], e1w2_ref[...])
    t2 = _dot(d2_ref[...], e2w2_ref[...])
    s1 = jnp.sum(t1 * t1)
    s2 = jnp.sum(t2 * t2)
    lane = lax.broadcasted_iota(jnp.int32, (1, 128), 1)
    contrib = jnp.where(lane == 0, s1, 0.0) + jnp.where(lane == 1, s2, 0.0)

    @pl.when(i == 0)
    def _():
        ssq_ref[...] = jnp.zeros_like(ssq_ref)
    ssq_ref[...] += contrib


def _final(r1, d1, r2, d2, ang, e1w2, e1w3, e1b2, e1b3, e2w2, e2w3, e2b2,
           e2b3):
    return pl.pallas_call(
        _final_body,
        grid=(E // BE,),
        in_specs=[
            pl.BlockSpec((BE, D), lambda i: (i, 0)),
            pl.BlockSpec((BE, D), lambda i: (i, 0)),
            pl.BlockSpec((BE, D), lambda i: (i, 0)),
            pl.BlockSpec((BE, D), lambda i: (i, 0)),
            pl.BlockSpec((BE, 1), lambda i: (i, 0)),
            pl.BlockSpec((D, D), lambda i: (0, 0)),
            pl.BlockSpec((D + 1, D), lambda i: (0, 0)),
            pl.BlockSpec((1, D), lambda i: (0, 0)),
            pl.BlockSpec((1, D), lambda i: (0, 0)),
            pl.BlockSpec((D, D), lambda i: (0, 0)),
            pl.BlockSpec((2 * D, OUT), lambda i: (0, 0)),
            pl.BlockSpec((1, D), lambda i: (0, 0)),
            pl.BlockSpec((1, OUT), lambda i: (0, 0)),
        ],
        out_specs=[
            pl.BlockSpec((BE, OUT), lambda i: (i, 0)),
            pl.BlockSpec((1, 128), lambda i: (0, 0)),
        ],
        out_shape=[
            jax.ShapeDtypeStruct((E, OUT), jnp.float32),
            jax.ShapeDtypeStruct((1, 128), jnp.float32),
        ],
    )(r1, d1, r2, d2, ang, e1w2, e1w3, e1b2, e1b3, e2w2, e2w3, e2b2, e2b3)


def kernel(node_features, edge_index, angles, gt_edges,
           nc1_W1, nc1_b1, nc1_W2, nc1_b2, nc2_W1, nc2_b1, nc2_W2, nc2_b2,
           ec1_W1, ec1_b1, ec1_W2, ec1_b2, ec1_W3, ec1_b3,
           ec2_W1, ec2_b1, ec2_W2, ec2_b2, ec2_W3, ec2_b3):
    src = edge_index[0]
    dst = edge_index[1]
    r2d = lambda b: b.reshape(1, -1)

    qa, qb = _prep(node_features, nc1_W1, r2d(nc1_b1))
    s1 = _sc_scatter_deg(qa, qb, src, dst)
    x1, p1a, p1b, q2a, q2b = _mid1(s1, nc1_W2, r2d(nc1_b2), ec1_W1,
                                   r2d(ec1_b1), nc2_W1, r2d(nc2_b1))
    r1, d1 = _sc_edge(p1a, p1b, src, dst)
    s2 = _sc_scatter_plain(q2a, q2b, src, dst)
    p2a, p2b = _mid2(s1, s2, x1, nc2_W2, r2d(nc2_b2), ec2_W1, r2d(ec2_b1))
    r2, d2 = _sc_edge(p2a, p2b, src, dst)
    e2, ssq = _final(r1, d1, r2, d2, angles, ec1_W2, ec1_W3, r2d(ec1_b2),
                     r2d(ec1_b3), ec2_W2, ec2_W3, r2d(ec2_b2), r2d(ec2_b3))
    side = ((ssq[0, 0] + ssq[0, 1]) / (E * D) * 0.5).reshape(1)
    return e2, side


# speedup vs baseline: 2.5090x; 1.0453x over previous
import functools

import jax
import jax.numpy as jnp
from jax import lax
from jax.experimental import pallas as pl
from jax.experimental.pallas import tpu as pltpu
from jax.experimental.pallas import tpu_sc as plsc

N = 10000
E = 320000
D = 128
OUT = 32

NC = 2
NS = 16
NW = NC * NS
EPW = E // NW
K = 80
NCHUNK = EPW // K
SROW = D + 16
NP = 10240
ZR = 80
RPT = NP // NS

_MESH = plsc.VectorSubcoreMesh(
    core_axis_name="c", subcore_axis_name="s", num_cores=NC, num_subcores=NS)


def _make_sc_scatter(width):
    ngrp = D // 16
    wgrp = width // 16

    @functools.partial(
        pl.kernel,
        out_type=jax.ShapeDtypeStruct((NC, NP, width), jnp.float32),
        mesh=_MESH,
        compiler_params=pltpu.CompilerParams(use_tc_tiling_on_sc=False),
        scratch_types=[
            pltpu.VMEM((K,), jnp.int32),
            pltpu.VMEM((K,), jnp.int32),
            pltpu.VMEM((K, D), jnp.float32),
            pltpu.VMEM((K, D), jnp.float32),
            pltpu.VMEM((K, width), jnp.float32),
            pltpu.VMEM_SHARED((NP, width), jnp.float32),
            pltpu.SemaphoreType.DMA,
            pltpu.SemaphoreType.DMA,
        ],
    )
    def kern(qa, qb, src, dst, out, idx_s, idx_d, abuf, bbuf, rbuf,
             s_sh, sem_a, sem_b):
        cid = lax.axis_index("c")
        sid = lax.axis_index("s")
        wid = sid * NC + cid

        zero16 = jnp.zeros((16,), jnp.float32)

        def zrow(r, carry):
            for g in range(wgrp):
                rbuf[r, pl.ds(g * 16, 16)] = zero16
            return carry
        lax.fori_loop(0, ZR, zrow, 0)
        for j in range(RPT // ZR):
            row0 = sid * RPT + j * ZR
            pltpu.sync_copy(rbuf, s_sh.at[pl.ds(row0, ZR)])

        if width > D:
            lane = lax.broadcasted_iota(jnp.int32, (16,), 0)
            onehot = jnp.where(lane == 0, 1.0, 0.0).astype(jnp.float32)
            def irow(r, carry):
                rbuf[r, pl.ds(D, 16)] = onehot
                return carry
            lax.fori_loop(0, K, irow, 0)

        plsc.subcore_barrier()

        def chunk(i, carry):
            base = wid * EPW + i * K
            pltpu.sync_copy(src.at[pl.ds(base, K)], idx_s)
            pltpu.sync_copy(dst.at[pl.ds(base, K)], idx_d)
            cp_a = pltpu.async_copy(qa.at[idx_s], abuf, sem_a)
            cp_b = pltpu.async_copy(qb.at[idx_d], bbuf, sem_b)
            cp_a.wait()
            cp_b.wait()

            def row(r, c2):
                for g in range(ngrp):
                    sl = pl.ds(g * 16, 16)
                    rbuf[r, sl] = jnp.maximum(abuf[r, sl] + bbuf[r, sl], 0.0)
                return c2
            lax.fori_loop(0, K, row, 0)
            pltpu.sync_copy(rbuf, s_sh.at[idx_d], add=True)
            return carry
        lax.fori_loop(0, NCHUNK, chunk, 0)

        plsc.subcore_barrier()

        for j in range(RPT // ZR):
            row0 = sid * RPT + j * ZR
            pltpu.sync_copy(s_sh.at[pl.ds(row0, ZR)], rbuf)
            pltpu.sync_copy(rbuf, out.at[cid, pl.ds(row0, ZR)])

    return kern


_sc_scatter_deg = _make_sc_scatter(SROW)
_sc_scatter_plain = _make_sc_scatter(D)


@functools.partial(
    pl.kernel,
    out_type=jax.ShapeDtypeStruct((E, 2 * D), jnp.float32),
    mesh=_MESH,
    compiler_params=pltpu.CompilerParams(use_tc_tiling_on_sc=False),
    scratch_types=[
        pltpu.VMEM((NCHUNK, K), jnp.int32),
        pltpu.VMEM((NCHUNK, K), jnp.int32),
        pltpu.VMEM((2, K, 2 * D), jnp.float32),
        pltpu.VMEM((2, K, 2 * D), jnp.float32),
        pltpu.VMEM((K, 2 * D), jnp.float32),
        pltpu.SemaphoreType.DMA,
        pltpu.SemaphoreType.DMA,
        pltpu.SemaphoreType.DMA,
    ],
)
def _sc_edge(tt, src2, dst2, rd_out, idx2_s, idx2_d, gs, gd, gout,
             sem_g0, sem_g1, sem_st):
    cid = lax.axis_index("c")
    sid = lax.axis_index("s")
    wid = sid * NC + cid
    ngrp = D // 16
    sems = (sem_g0, sem_g1)

    pltpu.sync_copy(src2.at[pl.ds(wid * NCHUNK, NCHUNK)], idx2_s)
    pltpu.sync_copy(dst2.at[pl.ds(wid * NCHUNK, NCHUNK)], idx2_d)

    def fire(i, b):
        cp1 = pltpu.async_copy(tt.at[idx2_s.at[i]], gs.at[b], sems[b])
        cp2 = pltpu.async_copy(tt.at[idx2_d.at[i]], gd.at[b], sems[b])
        return cp1, cp2

    def wait_g(i, b):
        pltpu.make_async_copy(tt.at[idx2_s.at[i]], gs.at[b], sems[b]).wait()
        pltpu.make_async_copy(tt.at[idx2_d.at[i]], gd.at[b], sems[b]).wait()

    def compute(b):
        gsb = gs.at[b]
        gdb = gd.at[b]

        def row(r, c2):
            for g in range(ngrp):
                lo = pl.ds(g * 16, 16)
                hi = pl.ds(D + g * 16, 16)
                ru = jnp.maximum(gsb[r, lo] + gdb[r, hi], 0.0)
                rv = jnp.maximum(gdb[r, lo] + gsb[r, hi], 0.0)
                gout[r, lo] = (ru + rv) * 0.5
                gout[r, hi] = ru - rv
            return c2
        lax.fori_loop(0, K, row, 0)

    def fire_store(i):
        return pltpu.async_copy(
            gout, rd_out.at[pl.ds(wid * EPW + i * K, K)], sem_st)

    def drain_store(i):
        pltpu.make_async_copy(
            gout, rd_out.at[pl.ds(wid * EPW + i * K, K)], sem_st).wait()

    fire(0, 0)
    fire(1, 1)
    wait_g(0, 0)
    compute(0)
    fire_store(0)
    fire(2, 0)

    def pair(j, carry):
        for (c_off, b) in ((1, 1), (2, 0)):
            c = 2 * j + c_off
            drain_store(c - 1)
            wait_g(c, b)
            compute(b)
            fire_store(c)
            fire(c + 2, b)
        return carry
    lax.fori_loop(0, 60, pair, 0)

    for c in (121, 122, 123, 124):
        b = c % 2
        drain_store(c - 1)
        wait_g(c, b)
        compute(b)
        fire_store(c)
        if c + 2 <= NCHUNK - 1:
            fire(c + 2, b)
    drain_store(NCHUNK - 1)


BN = 2000
BE = 3200


def _dot(a, b):
    return jax.lax.dot_general(a, b, (((1,), (0,)), ((), ())),
                               preferred_element_type=jnp.float32)


def _prep_body(x_ref, w_ref, b_ref, qa_ref, qb_ref):
    x = x_ref[...]
    qa_ref[...] = _dot(x, w_ref[:D])
    qb_ref[...] = _dot(x, w_ref[D:]) + b_ref[...]


def _prep(x, w1, b1):
    return pl.pallas_call(
        _prep_body,
        grid=(N // BN,),
        in_specs=[
            pl.BlockSpec((BN, D), lambda i: (i, 0)),
            pl.BlockSpec((2 * D, D), lambda i: (0, 0)),
            pl.BlockSpec((1, D), lambda i: (0, 0)),
        ],
        out_specs=[pl.BlockSpec((BN, D), lambda i: (i, 0))] * 2,
        out_shape=[jax.ShapeDtypeStruct((N, D), jnp.float32)] * 2,
    )(x, w1, b1)


def _mid1_body(sp_ref, w2_ref, b2_ref, ew1_ref, eb1_ref, nw1_ref, nb1_ref,
               x1_ref, t1_ref, q2a_ref, q2b_ref):
    s = sp_ref[0] + sp_ref[1]
    deg = s[:, D:D + 1]
    agg = _dot(s[:, :D], w2_ref[...]) + deg * b2_ref[...]
    x1 = jnp.maximum(agg / jnp.maximum(deg, 1.0), 0.0)
    x1_ref[...] = x1
    t1_ref[:, :D] = _dot(x1, ew1_ref[:D])
    t1_ref[:, D:] = _dot(x1, ew1_ref[D:]) + eb1_ref[...]
    mu = jnp.mean(x1, axis=1, keepdims=True)
    var = jnp.mean((x1 - mu) ** 2, axis=1, keepdims=True)
    xn = (x1 - mu) / (jnp.sqrt(var) + 1e-6)
    q2a_ref[...] = _dot(xn, nw1_ref[:D])
    q2b_ref[...] = _dot(xn, nw1_ref[D:]) + nb1_ref[...]


def _mid1(s1, w2, b2, ew1, eb1, nw1, nb1):
    return pl.pallas_call(
        _mid1_body,
        grid=(N // BN,),
        in_specs=[
            pl.BlockSpec((NC, BN, SROW), lambda i: (0, i, 0)),
            pl.BlockSpec((D, D), lambda i: (0, 0)),
            pl.BlockSpec((1, D), lambda i: (0, 0)),
            pl.BlockSpec((2 * D, D), lambda i: (0, 0)),
            pl.BlockSpec((1, D), lambda i: (0, 0)),
            pl.BlockSpec((2 * D, D), lambda i: (0, 0)),
            pl.BlockSpec((1, D), lambda i: (0, 0)),
        ],
        out_specs=[
            pl.BlockSpec((BN, D), lambda i: (i, 0)),
            pl.BlockSpec((BN, 2 * D), lambda i: (i, 0)),
            pl.BlockSpec((BN, D), lambda i: (i, 0)),
            pl.BlockSpec((BN, D), lambda i: (i, 0)),
        ],
        out_shape=[
            jax.ShapeDtypeStruct((N, D), jnp.float32),
            jax.ShapeDtypeStruct((N, 2 * D), jnp.float32),
            jax.ShapeDtypeStruct((N, D), jnp.float32),
            jax.ShapeDtypeStruct((N, D), jnp.float32),
        ],
    )(s1, w2, b2, ew1, eb1, nw1, nb1)


def _mid2_body(s1p_ref, s2p_ref, x1_ref, w2_ref, b2_ref, ew1_ref, eb1_ref,
               t2_ref):
    deg = s1p_ref[0, :, D:D + 1] + s1p_ref[1, :, D:D + 1]
    s2 = s2p_ref[0] + s2p_ref[1]
    agg = _dot(s2, w2_ref[...]) + deg * b2_ref[...]
    x2 = x1_ref[...] + jnp.maximum(agg / jnp.maximum(deg, 1.0), 0.0)
    t2_ref[:, :D] = _dot(x2, ew1_ref[:D])
    t2_ref[:, D:] = _dot(x2, ew1_ref[D:]) + eb1_ref[...]


def _mid2(s1, s2, x1, w2, b2, ew1, eb1):
    return pl.pallas_call(
        _mid2_body,
        grid=(N // BN,),
        in_specs=[
            pl.BlockSpec((NC, BN, SROW), lambda i: (0, i, 0)),
            pl.BlockSpec((NC, BN, D), lambda i: (0, i, 0)),
            pl.BlockSpec((BN, D), lambda i: (i, 0)),
            pl.BlockSpec((D, D), lambda i: (0, 0)),
            pl.BlockSpec((1, D), lambda i: (0, 0)),
            pl.BlockSpec((2 * D, D), lambda i: (0, 0)),
            pl.BlockSpec((1, D), lambda i: (0, 0)),
        ],
        out_specs=[pl.BlockSpec((BN, 2 * D), lambda i: (i, 0))],
        out_shape=[jax.ShapeDtypeStruct((N, 2 * D), jnp.float32)],
    )(s1, s2, x1, w2, b2, ew1, eb1)[0]


def _final_body(r1_ref, d1_ref, r2_ref, d2_ref, ang_ref,
                e1w2_ref, e1w3_ref, e1b2_ref, e1b3_ref,
                e2w2_ref, e2w3_ref, e2b2_ref, e2b3_ref,
                e2_ref, ssq_ref):
    i = pl.program_id(0)
    w3a1 = e1w3_ref[:D]
    v1 = e1w3_ref[D:D + 1]
    m1 = _dot(e1w2_ref[...], w3a1)
    c1 = _dot(e1b2_ref[...], w3a1) + e1b3_ref[...]
    w3a2 = e2w3_ref[:D]
    w3b2 = e2w3_ref[D:]
    a2 = _dot(e2w2_ref[...], w3a2)
    a1 = _dot(m1, w3b2)
    wv = _dot(v1, w3b2)
    cc = _dot(c1, w3b2) + _dot(e2b2_ref[...], w3a2) + e2b3_ref[...]

    e2 = (_dot(r2_ref[...], a2) + _dot(r1_ref[...], a1)
          + ang_ref[...] * wv + cc)
    e2_ref[...] = e2

    t1 = _dot(d1_ref[...], e1w2_ref[...])
    t2 = _dot(d2_ref[...], e2w2_ref[...])
    s1 = jnp.sum(t1 * t1)
    s2 = jnp.sum(t2 * t2)
    lane = lax.broadcasted_iota(jnp.int32, (1, 128), 1)
    contrib = jnp.where(lane == 0, s1, 0.0) + jnp.where(lane == 1, s2, 0.0)

    @pl.when(i == 0)
    def _():
        ssq_ref[...] = jnp.zeros_like(ssq_ref)
    ssq_ref[...] += contrib


def _final(r1, d1, r2, d2, ang, e1w2, e1w3, e1b2, e1b3, e2w2, e2w3, e2b2,
           e2b3):
    return pl.pallas_call(
        _final_body,
        grid=(E // BE,),
        in_specs=[
            pl.BlockSpec((BE, D), lambda i: (i, 0)),
            pl.BlockSpec((BE, D), lambda i: (i, 1)),
            pl.BlockSpec((BE, D), lambda i: (i, 0)),
            pl.BlockSpec((BE, D), lambda i: (i, 1)),
            pl.BlockSpec((BE, 1), lambda i: (i, 0)),
            pl.BlockSpec((D, D), lambda i: (0, 0)),
            pl.BlockSpec((D + 1, D), lambda i: (0, 0)),
            pl.BlockSpec((1, D), lambda i: (0, 0)),
            pl.BlockSpec((1, D), lambda i: (0, 0)),
            pl.BlockSpec((D, D), lambda i: (0, 0)),
            pl.BlockSpec((2 * D, OUT), lambda i: (0, 0)),
            pl.BlockSpec((1, D), lambda i: (0, 0)),
            pl.BlockSpec((1, OUT), lambda i: (0, 0)),
        ],
        out_specs=[
            pl.BlockSpec((BE, OUT), lambda i: (i, 0)),
            pl.BlockSpec((1, 128), lambda i: (0, 0)),
        ],
        out_shape=[
            jax.ShapeDtypeStruct((E, OUT), jnp.float32),
            jax.ShapeDtypeStruct((1, 128), jnp.float32),
        ],
    )(r1, d1, r2, d2, ang, e1w2, e1w3, e1b2, e1b3, e2w2, e2w3, e2b2, e2b3)


def kernel(node_features, edge_index, angles, gt_edges,
           nc1_W1, nc1_b1, nc1_W2, nc1_b2, nc2_W1, nc2_b1, nc2_W2, nc2_b2,
           ec1_W1, ec1_b1, ec1_W2, ec1_b2, ec1_W3, ec1_b3,
           ec2_W1, ec2_b1, ec2_W2, ec2_b2, ec2_W3, ec2_b3):
    src = edge_index[0]
    dst = edge_index[1]
    src2 = src.reshape(E // K, K)
    dst2 = dst.reshape(E // K, K)
    r2d = lambda b: b.reshape(1, -1)

    qa, qb = _prep(node_features, nc1_W1, r2d(nc1_b1))
    s1 = _sc_scatter_deg(qa, qb, src, dst)
    x1, t1, q2a, q2b = _mid1(s1, nc1_W2, r2d(nc1_b2), ec1_W1,
                             r2d(ec1_b1), nc2_W1, r2d(nc2_b1))
    rd1 = _sc_edge(t1, src2, dst2)
    s2 = _sc_scatter_plain(q2a, q2b, src, dst)
    t2 = _mid2(s1, s2, x1, nc2_W2, r2d(nc2_b2), ec2_W1, r2d(ec2_b1))
    rd2 = _sc_edge(t2, src2, dst2)
    e2, ssq = _final(rd1, rd1, rd2, rd2, angles, ec1_W2, ec1_W3, r2d(ec1_b2),
                     r2d(ec1_b3), ec2_W2, ec2_W3, r2d(ec2_b2), r2d(ec2_b3))
    side = ((ssq[0, 0] + ssq[0, 1]) / (E * D) * 0.5).reshape(1)
    return e2, side


# speedup vs baseline: 3.4957x; 1.3933x over previous
import functools

import jax
import jax.numpy as jnp
from jax import lax
from jax.experimental import pallas as pl
from jax.experimental.pallas import tpu as pltpu
from jax.experimental.pallas import tpu_sc as plsc

N = 10000
E = 320000
D = 128
OUT = 32

NC = 2
NS = 16
NW = NC * NS
EPW = E // NW
K = 80
NCHUNK = EPW // K
KS = 80
NCHS = EPW // KS
SROW = D + 16
NP = 10240
ZR = 80
RPT = NP // NS

_MESH = plsc.VectorSubcoreMesh(
    core_axis_name="c", subcore_axis_name="s", num_cores=NC, num_subcores=NS)


def _make_sc_scatter(with_deg):
    ngrp = D // 16

    out_type = [jax.ShapeDtypeStruct((NC, NP, D), jnp.float32)]
    scratch = [
        pltpu.VMEM((KS,), jnp.int32),
        pltpu.VMEM((KS,), jnp.int32),
        pltpu.VMEM((KS,), jnp.int32),
        pltpu.VMEM((KS,), jnp.int32),
        pltpu.VMEM((KS,), jnp.int32),
        pltpu.VMEM((KS,), jnp.int32),
        pltpu.VMEM((KS,), jnp.int32),
        pltpu.VMEM((KS,), jnp.int32),
        pltpu.VMEM((KS, D), jnp.float32),
        pltpu.VMEM((KS, D), jnp.float32),
        pltpu.VMEM((KS, D), jnp.float32),
        pltpu.VMEM((KS, D), jnp.float32),
        pltpu.VMEM_SHARED((NP, D), jnp.float32),
        pltpu.SemaphoreType.DMA,
        pltpu.SemaphoreType.DMA,
        pltpu.SemaphoreType.DMA,
        pltpu.SemaphoreType.DMA,
    ]

    @functools.partial(
        pl.kernel,
        out_type=out_type,
        mesh=_MESH,
        compiler_params=pltpu.CompilerParams(use_tc_tiling_on_sc=False),
        scratch_types=scratch,
    )
    def kern(qa, qb, src2, dst2, *rest):
        (out, ixs0, ixs1, ixs2, ixs3, ixd0, ixd1, ixd2, ixd3,
         ab_0, ab_1, bb_0, bb_1, s_sh, sem_i, sem_g0, sem_g1,
         sem_s) = rest
        idxs = (ixs0, ixs1, ixs2, ixs3)
        idxd = (ixd0, ixd1, ixd2, ixd3)
        abufs = (ab_0, ab_1)
        bbufs = (bb_0, bb_1)
        cid = lax.axis_index("c")
        sid = lax.axis_index("s")
        wid = sid * NC + cid
        gsems = (sem_g0, sem_g1)
        zero16 = jnp.zeros((16,), jnp.float32)

        ab0 = ab_0
        def zrow(r, carry):
            for g in range(ngrp):
                ab0[r, pl.ds(g * 16, 16)] = zero16
            return carry
        lax.fori_loop(0, KS, zrow, 0)
        for j in range(RPT // KS):
            row0 = sid * RPT + j * KS
            pltpu.sync_copy(ab0, s_sh.at[pl.ds(row0, KS)])
        plsc.subcore_barrier()

        def fire_idx(i, s4):
            pltpu.async_copy(src2.at[wid * NCHS + i], idxs[s4], sem_i)
            pltpu.async_copy(dst2.at[wid * NCHS + i], idxd[s4], sem_i)

        def wait_idx(i, s4):
            pltpu.make_async_copy(
                src2.at[wid * NCHS + i], idxs[s4], sem_i).wait()
            pltpu.make_async_copy(
                dst2.at[wid * NCHS + i], idxd[s4], sem_i).wait()

        def fire_g(s4, b):
            pltpu.async_copy(qa.at[idxs[s4]], abufs[b], gsems[b])
            pltpu.async_copy(qb.at[idxd[s4]], bbufs[b], gsems[b])

        def wait_g(s4, b):
            pltpu.make_async_copy(qa.at[idxs[s4]], abufs[b],
                                  gsems[b]).wait()
            pltpu.make_async_copy(qb.at[idxd[s4]], bbufs[b],
                                  gsems[b]).wait()

        def compute(b):
            ab = abufs[b]
            bb = bbufs[b]
            def row(r, c2):
                for g in range(ngrp):
                    sl = pl.ds(g * 16, 16)
                    ab[r, sl] = jnp.maximum(ab[r, sl] + bb[r, sl], 0.0)
                return c2
            lax.fori_loop(0, KS, row, 0)

        def fire_s(s4, b):
            pltpu.async_copy(abufs[b], s_sh.at[idxd[s4]], sem_s, add=True)

        def drain_s(s4, b):
            pltpu.make_async_copy(abufs[b], s_sh.at[idxd[s4]],
                                  sem_s).wait()

        fire_idx(0, 0)
        fire_idx(1, 1)
        fire_idx(2, 2)
        fire_idx(3, 3)
        wait_idx(0, 0)
        fire_g(0, 0)
        wait_g(0, 0)
        compute(0)
        fire_s(0, 0)
        wait_idx(1, 1)
        fire_g(1, 1)

        NQ = (NCHS - 6) // 4
        def quad(j, carry):
            for c_off in (1, 2, 3, 4):
                c = 4 * j + c_off
                b = c_off % 2
                drain_s((c_off - 1) % 4, 1 - b)
                fire_idx(c + 3, (c_off + 3) % 4)
                wait_idx(c + 1, (c_off + 1) % 4)
                fire_g((c_off + 1) % 4, 1 - b)
                wait_g(c_off % 4, b)
                compute(b)
                fire_s(c_off % 4, b)
            return carry
        lax.fori_loop(0, NQ, quad, 0)

        for c in range(4 * NQ + 1, NCHS):
            b = c % 2
            drain_s((c - 1) % 4, 1 - b)
            if c + 3 <= NCHS - 1:
                fire_idx(c + 3, (c + 3) % 4)
            if c + 1 <= NCHS - 1:
                wait_idx(c + 1, (c + 1) % 4)
                fire_g((c + 1) % 4, 1 - b)
            wait_g(c % 4, b)
            compute(b)
            fire_s(c % 4, b)
        drain_s((NCHS - 1) % 4, (NCHS - 1) % 2)

        plsc.subcore_barrier()

        for j in range(RPT // KS):
            row0 = sid * RPT + j * KS
            pltpu.sync_copy(s_sh.at[pl.ds(row0, KS)], ab0)
            pltpu.sync_copy(ab0, out.at[cid, pl.ds(row0, KS)])

    return kern


_sc_scatter = _make_sc_scatter(False)


@functools.partial(
    pl.kernel,
    out_type=jax.ShapeDtypeStruct((NW, NP), jnp.float32),
    mesh=_MESH,
    compiler_params=pltpu.CompilerParams(use_tc_tiling_on_sc=False,
                                         needs_layout_passes=False),
    scratch_types=[
        pltpu.VMEM((K,), jnp.int32),
        pltpu.VMEM((K,), jnp.int32),
        pltpu.VMEM((K,), jnp.int32),
        pltpu.VMEM((K,), jnp.int32),
        pltpu.VMEM((NP,), jnp.float32),
        pltpu.SemaphoreType.DMA,
    ],
)
def _sc_deg(dst2, out, ix0, ix1, ix2, ix3, hist, sem_i):
    cid = lax.axis_index("c")
    sid = lax.axis_index("s")
    wid = sid * NC + cid
    idxs = (ix0, ix1, ix2, ix3)
    zero16 = jnp.zeros((16,), jnp.float32)
    ones16 = jnp.ones((16,), jnp.float32)

    def hzero(r, carry):
        hist[pl.ds(r * 16, 16)] = zero16
        return carry
    lax.fori_loop(0, NP // 16, hzero, 0)

    def fire(i, s4):
        pltpu.async_copy(dst2.at[wid * NCHUNK + i], idxs[s4], sem_i)

    def wait(i, s4):
        pltpu.make_async_copy(dst2.at[wid * NCHUNK + i], idxs[s4],
                              sem_i).wait()

    def process(s4):
        for g in range(K // 16):
            v = idxs[s4][pl.ds(g * 16, 16)]
            plsc.addupdate_scatter(hist, [v], ones16)

    fire(0, 0)
    fire(1, 1)
    fire(2, 2)
    fire(3, 3)

    def quad(j, carry):
        for c_off in (0, 1, 2, 3):
            c = 4 * j + c_off
            wait(c, c_off)
            process(c_off)
            fire(c + 4, c_off)
        return carry
    lax.fori_loop(0, (NCHUNK - 5) // 4, quad, 0)

    for c in range(4 * ((NCHUNK - 5) // 4), NCHUNK):
        s4 = c % 4
        wait(c, s4)
        process(s4)
        if c + 4 <= NCHUNK - 1:
            fire(c + 4, s4)

    pltpu.sync_copy(hist, out.at[wid])


@functools.partial(
    pl.kernel,
    out_type=jax.ShapeDtypeStruct((E, 2 * D), jnp.float32),
    mesh=_MESH,
    compiler_params=pltpu.CompilerParams(use_tc_tiling_on_sc=False),
    scratch_types=[
        pltpu.VMEM((NCHUNK, K), jnp.int32),
        pltpu.VMEM((NCHUNK, K), jnp.int32),
        pltpu.VMEM((2, K, 2 * D), jnp.float32),
        pltpu.VMEM((2, K, 2 * D), jnp.float32),
        pltpu.VMEM((K, 2 * D), jnp.float32),
        pltpu.SemaphoreType.DMA,
        pltpu.SemaphoreType.DMA,
        pltpu.SemaphoreType.DMA,
    ],
)
def _sc_edge(tt, src2, dst2, rd_out, idx2_s, idx2_d, gs, gd, gout,
             sem_g0, sem_g1, sem_st):
    cid = lax.axis_index("c")
    sid = lax.axis_index("s")
    wid = sid * NC + cid
    ngrp = D // 16
    sems = (sem_g0, sem_g1)

    pltpu.sync_copy(src2.at[pl.ds(wid * NCHUNK, NCHUNK)], idx2_s)
    pltpu.sync_copy(dst2.at[pl.ds(wid * NCHUNK, NCHUNK)], idx2_d)

    def fire(i, b):
        cp1 = pltpu.async_copy(tt.at[idx2_s.at[i]], gs.at[b], sems[b])
        cp2 = pltpu.async_copy(tt.at[idx2_d.at[i]], gd.at[b], sems[b])
        return cp1, cp2

    def wait_g(i, b):
        pltpu.make_async_copy(tt.at[idx2_s.at[i]], gs.at[b], sems[b]).wait()
        pltpu.make_async_copy(tt.at[idx2_d.at[i]], gd.at[b], sems[b]).wait()

    def compute(b):
        gsb = gs.at[b]
        gdb = gd.at[b]

        def row(r, c2):
            for g in range(ngrp):
                lo = pl.ds(g * 16, 16)
                hi = pl.ds(D + g * 16, 16)
                ru = jnp.maximum(gsb[r, lo] + gdb[r, hi], 0.0)
                rv = jnp.maximum(gdb[r, lo] + gsb[r, hi], 0.0)
                gout[r, lo] = (ru + rv) * 0.5
                gout[r, hi] = ru - rv
            return c2
        lax.fori_loop(0, K, row, 0)

    def fire_store(i):
        return pltpu.async_copy(
            gout, rd_out.at[pl.ds(wid * EPW + i * K, K)], sem_st)

    def drain_store(i):
        pltpu.make_async_copy(
            gout, rd_out.at[pl.ds(wid * EPW + i * K, K)], sem_st).wait()

    fire(0, 0)
    fire(1, 1)
    wait_g(0, 0)
    compute(0)
    fire_store(0)
    fire(2, 0)

    def pair(j, carry):
        for (c_off, b) in ((1, 1), (2, 0)):
            c = 2 * j + c_off
            drain_store(c - 1)
            wait_g(c, b)
            compute(b)
            fire_store(c)
            fire(c + 2, b)
        return carry
    lax.fori_loop(0, 60, pair, 0)

    for c in (121, 122, 123, 124):
        b = c % 2
        drain_store(c - 1)
        wait_g(c, b)
        compute(b)
        fire_store(c)
        if c + 2 <= NCHUNK - 1:
            fire(c + 2, b)
    drain_store(NCHUNK - 1)


BN = 2000
BM = 1280
BE = 3200


def _dot(a, b):
    return jax.lax.dot_general(a, b, (((1,), (0,)), ((), ())),
                               preferred_element_type=jnp.float32)


def _prep_body(x_ref, w_ref, b_ref, qa_ref, qb_ref):
    x = x_ref[...]
    qa_ref[...] = _dot(x, w_ref[:D])
    qb_ref[...] = _dot(x, w_ref[D:]) + b_ref[...]


def _prep(x, w1, b1):
    return pl.pallas_call(
        _prep_body,
        grid=(N // BN,),
        in_specs=[
            pl.BlockSpec((BN, D), lambda i: (i, 0)),
            pl.BlockSpec((2 * D, D), lambda i: (0, 0)),
            pl.BlockSpec((1, D), lambda i: (0, 0)),
        ],
        out_specs=[pl.BlockSpec((BN, D), lambda i: (i, 0))] * 2,
        out_shape=[jax.ShapeDtypeStruct((N, D), jnp.float32)] * 2,
    )(x, w1, b1)


def _mid1_body(sp_ref, dp_ref, w2_ref, b2_ref, ew1_ref, eb1_ref, nw1_ref,
               nb1_ref, x1_ref, t1_ref, q2a_ref, q2b_ref):
    s = sp_ref[0] + sp_ref[1]
    deg = jax.lax.dot_general(dp_ref[...], jnp.ones((NW, 1), jnp.float32),
                              (((0,), (0,)), ((), ())),
                              preferred_element_type=jnp.float32)
    agg = _dot(s, w2_ref[...]) + deg * b2_ref[...]
    x1 = jnp.maximum(agg / jnp.maximum(deg, 1.0), 0.0)
    x1_ref[...] = x1
    t1_ref[:, :D] = _dot(x1, ew1_ref[:D])
    t1_ref[:, D:] = _dot(x1, ew1_ref[D:]) + eb1_ref[...]
    mu = jnp.mean(x1, axis=1, keepdims=True)
    var = jnp.mean((x1 - mu) ** 2, axis=1, keepdims=True)
    xn = (x1 - mu) / (jnp.sqrt(var) + 1e-6)
    q2a_ref[...] = _dot(xn, nw1_ref[:D])
    q2b_ref[...] = _dot(xn, nw1_ref[D:]) + nb1_ref[...]


def _mid1(s1, degp, w2, b2, ew1, eb1, nw1, nb1):
    return pl.pallas_call(
        _mid1_body,
        grid=(NP // BM,),
        in_specs=[
            pl.BlockSpec((NC, BM, D), lambda i: (0, i, 0)),
            pl.BlockSpec((NW, BM), lambda i: (0, i)),
            pl.BlockSpec((D, D), lambda i: (0, 0)),
            pl.BlockSpec((1, D), lambda i: (0, 0)),
            pl.BlockSpec((2 * D, D), lambda i: (0, 0)),
            pl.BlockSpec((1, D), lambda i: (0, 0)),
            pl.BlockSpec((2 * D, D), lambda i: (0, 0)),
            pl.BlockSpec((1, D), lambda i: (0, 0)),
        ],
        out_specs=[
            pl.BlockSpec((BM, D), lambda i: (i, 0)),
            pl.BlockSpec((BM, 2 * D), lambda i: (i, 0)),
            pl.BlockSpec((BM, D), lambda i: (i, 0)),
            pl.BlockSpec((BM, D), lambda i: (i, 0)),
        ],
        out_shape=[
            jax.ShapeDtypeStruct((NP, D), jnp.float32),
            jax.ShapeDtypeStruct((NP, 2 * D), jnp.float32),
            jax.ShapeDtypeStruct((NP, D), jnp.float32),
            jax.ShapeDtypeStruct((NP, D), jnp.float32),
        ],
    )(s1, degp, w2, b2, ew1, eb1, nw1, nb1)


def _mid2_body(dp_ref, s2p_ref, x1_ref, w2_ref, b2_ref, ew1_ref, eb1_ref,
               t2_ref):
    deg = jax.lax.dot_general(dp_ref[...], jnp.ones((NW, 1), jnp.float32),
                              (((0,), (0,)), ((), ())),
                              preferred_element_type=jnp.float32)
    s2 = s2p_ref[0] + s2p_ref[1]
    agg = _dot(s2, w2_ref[...]) + deg * b2_ref[...]
    x2 = x1_ref[...] + jnp.maximum(agg / jnp.maximum(deg, 1.0), 0.0)
    t2_ref[:, :D] = _dot(x2, ew1_ref[:D])
    t2_ref[:, D:] = _dot(x2, ew1_ref[D:]) + eb1_ref[...]


def _mid2(degp, s2, x1, w2, b2, ew1, eb1):
    return pl.pallas_call(
        _mid2_body,
        grid=(NP // BM,),
        in_specs=[
            pl.BlockSpec((NW, BM), lambda i: (0, i)),
            pl.BlockSpec((NC, BM, D), lambda i: (0, i, 0)),
            pl.BlockSpec((BM, D), lambda i: (i, 0)),
            pl.BlockSpec((D, D), lambda i: (0, 0)),
            pl.BlockSpec((1, D), lambda i: (0, 0)),
            pl.BlockSpec((2 * D, D), lambda i: (0, 0)),
            pl.BlockSpec((1, D), lambda i: (0, 0)),
        ],
        out_specs=[pl.BlockSpec((BM, 2 * D), lambda i: (i, 0))],
        out_shape=[jax.ShapeDtypeStruct((NP, 2 * D), jnp.float32)],
    )(degp, s2, x1, w2, b2, ew1, eb1)[0]


def _final_body(r1_ref, d1_ref, r2_ref, d2_ref, ang_ref,
                e1w2_ref, e1w3_ref, e1b2_ref, e1b3_ref,
                e2w2_ref, e2w3_ref, e2b2_ref, e2b3_ref,
                e2_ref, ssq_ref):
    i = pl.program_id(0)
    w3a1 = e1w3_ref[:D]
    v1 = e1w3_ref[D:D + 1]
    m1 = _dot(e1w2_ref[...], w3a1)
    c1 = _dot(e1b2_ref[...], w3a1) + e1b3_ref[...]
    w3a2 = e2w3_ref[:D]
    w3b2 = e2w3_ref[D:]
    a2 = _dot(e2w2_ref[...], w3a2)
    a1 = _dot(m1, w3b2)
    wv = _dot(v1, w3b2)
    cc = _dot(c1, w3b2) + _dot(e2b2_ref[...], w3a2) + e2b3_ref[...]

    e2 = (_dot(r2_ref[...], a2) + _dot(r1_ref[...], a1)
          + ang_ref[...] * wv + cc)
    e2_ref[...] = e2

    t1 = _dot(d1_ref[...], e1w2_ref[...])
    t2 = _dot(d2_ref[...], e2w2_ref[...])
    s1 = jnp.sum(t1 * t1)
    s2 = jnp.sum(t2 * t2)
    lane = lax.broadcasted_iota(jnp.int32, (1, 128), 1)
    contrib = jnp.where(lane == 0, s1, 0.0) + jnp.where(lane == 1, s2, 0.0)

    @pl.when(i == 0)
    def _():
        ssq_ref[...] = jnp.zeros_like(ssq_ref)
    ssq_ref[...] += contrib


def _final(r1, d1, r2, d2, ang, e1w2, e1w3, e1b2, e1b3, e2w2, e2w3, e2b2,
           e2b3):
    return pl.pallas_call(
        _final_body,
        grid=(E // BE,),
        in_specs=[
            pl.BlockSpec((BE, D), lambda i: (i, 0)),
            pl.BlockSpec((BE, D), lambda i: (i, 1)),
            pl.BlockSpec((BE, D), lambda i: (i, 0)),
            pl.BlockSpec((BE, D), lambda i: (i, 1)),
            pl.BlockSpec((BE, 1), lambda i: (i, 0)),
            pl.BlockSpec((D, D), lambda i: (0, 0)),
            pl.BlockSpec((D + 1, D), lambda i: (0, 0)),
            pl.BlockSpec((1, D), lambda i: (0, 0)),
            pl.BlockSpec((1, D), lambda i: (0, 0)),
            pl.BlockSpec((D, D), lambda i: (0, 0)),
            pl.BlockSpec((2 * D, OUT), lambda i: (0, 0)),
            pl.BlockSpec((1, D), lambda i: (0, 0)),
            pl.BlockSpec((1, OUT), lambda i: (0, 0)),
        ],
        out_specs=[
            pl.BlockSpec((BE, OUT), lambda i: (i, 0)),
            pl.BlockSpec((1, 128), lambda i: (0, 0)),
        ],
        out_shape=[
            jax.ShapeDtypeStruct((E, OUT), jnp.float32),
            jax.ShapeDtypeStruct((1, 128), jnp.float32),
        ],
    )(r1, d1, r2, d2, ang, e1w2, e1w3, e1b2, e1b3, e2w2, e2w3, e2b2, e2b3)


def kernel(node_features, edge_index, angles, gt_edges,
           nc1_W1, nc1_b1, nc1_W2, nc1_b2, nc2_W1, nc2_b1, nc2_W2, nc2_b2,
           ec1_W1, ec1_b1, ec1_W2, ec1_b2, ec1_W3, ec1_b3,
           ec2_W1, ec2_b1, ec2_W2, ec2_b2, ec2_W3, ec2_b3):
    src = edge_index[0]
    dst = edge_index[1]
    src2 = src.reshape(E // K, K)
    dst2 = dst.reshape(E // K, K)
    r2d = lambda b: b.reshape(1, -1)

    qa, qb = _prep(node_features, nc1_W1, r2d(nc1_b1))
    s1 = _sc_scatter(qa, qb, src2, dst2)[0]
    degp = _sc_deg(dst2)
    x1, t1, q2a, q2b = _mid1(s1, degp, nc1_W2, r2d(nc1_b2), ec1_W1,
                             r2d(ec1_b1), nc2_W1, r2d(nc2_b1))
    rd1 = _sc_edge(t1, src2, dst2)
    s2 = _sc_scatter(q2a, q2b, src2, dst2)[0]
    t2 = _mid2(degp, s2, x1, nc2_W2, r2d(nc2_b2), ec2_W1, r2d(ec2_b1))
    rd2 = _sc_edge(t2, src2, dst2)
    e2, ssq = _final(rd1, rd1, rd2, rd2, angles, ec1_W2, ec1_W3, r2d(ec1_b2),
                     r2d(ec1_b3), ec2_W2, ec2_W3, r2d(ec2_b2), r2d(ec2_b3))
    side = ((ssq[0, 0] + ssq[0, 1]) / (E * D) * 0.5).reshape(1)
    return e2, side


# speedup vs baseline: 3.8543x; 1.1026x over previous
import functools

import jax
import jax.numpy as jnp
from jax import lax
from jax.experimental import pallas as pl
from jax.experimental.pallas import tpu as pltpu
from jax.experimental.pallas import tpu_sc as plsc

N = 10000
E = 320000
D = 128
OUT = 32

NC = 2
NS = 16
NW = NC * NS
EPW = E // NW
K = 80
NCHUNK = EPW // K
KS = 80
NCHS = EPW // KS
SROW = D + 16
NP = 10240
ZR = 80
RPT = NP // NS

_MESH = plsc.VectorSubcoreMesh(
    core_axis_name="c", subcore_axis_name="s", num_cores=NC, num_subcores=NS)


def _make_sc_scatter(with_deg):
    ngrp = D // 16

    out_type = [jax.ShapeDtypeStruct((NC, NP, D), jnp.float32)]
    scratch = [
        pltpu.VMEM((KS,), jnp.int32),
        pltpu.VMEM((KS,), jnp.int32),
        pltpu.VMEM((KS,), jnp.int32),
        pltpu.VMEM((KS,), jnp.int32),
        pltpu.VMEM((KS,), jnp.int32),
        pltpu.VMEM((KS,), jnp.int32),
        pltpu.VMEM((KS,), jnp.int32),
        pltpu.VMEM((KS,), jnp.int32),
        pltpu.VMEM((KS, D), jnp.float32),
        pltpu.VMEM((KS, D), jnp.float32),
        pltpu.VMEM((KS, D), jnp.float32),
        pltpu.VMEM((KS, D), jnp.float32),
        pltpu.VMEM_SHARED((NP, D), jnp.float32),
        pltpu.SemaphoreType.DMA,
        pltpu.SemaphoreType.DMA,
        pltpu.SemaphoreType.DMA,
        pltpu.SemaphoreType.DMA,
    ]

    @functools.partial(
        pl.kernel,
        out_type=out_type,
        mesh=_MESH,
        compiler_params=pltpu.CompilerParams(use_tc_tiling_on_sc=False),
        scratch_types=scratch,
    )
    def kern(qa, qb, src2, dst2, *rest):
        (out, ixs0, ixs1, ixs2, ixs3, ixd0, ixd1, ixd2, ixd3,
         ab_0, ab_1, bb_0, bb_1, s_sh, sem_i, sem_g0, sem_g1,
         sem_s) = rest
        idxs = (ixs0, ixs1, ixs2, ixs3)
        idxd = (ixd0, ixd1, ixd2, ixd3)
        abufs = (ab_0, ab_1)
        bbufs = (bb_0, bb_1)
        cid = lax.axis_index("c")
        sid = lax.axis_index("s")
        wid = sid * NC + cid
        gsems = (sem_g0, sem_g1)
        zero16 = jnp.zeros((16,), jnp.float32)

        ab0 = ab_0
        def zrow(r, carry):
            for g in range(ngrp):
                ab0[r, pl.ds(g * 16, 16)] = zero16
            return carry
        lax.fori_loop(0, KS, zrow, 0)
        for j in range(RPT // KS):
            row0 = sid * RPT + j * KS
            pltpu.sync_copy(ab0, s_sh.at[pl.ds(row0, KS)])
        plsc.subcore_barrier()

        def fire_idx(i, s4):
            pltpu.async_copy(src2.at[wid * NCHS + i], idxs[s4], sem_i)
            pltpu.async_copy(dst2.at[wid * NCHS + i], idxd[s4], sem_i)

        def wait_idx(i, s4):
            pltpu.make_async_copy(
                src2.at[wid * NCHS + i], idxs[s4], sem_i).wait()
            pltpu.make_async_copy(
                dst2.at[wid * NCHS + i], idxd[s4], sem_i).wait()

        def fire_g(s4, b):
            pltpu.async_copy(qa.at[idxs[s4]], abufs[b], gsems[b])
            pltpu.async_copy(qb.at[idxd[s4]], bbufs[b], gsems[b])

        def wait_g(s4, b):
            pltpu.make_async_copy(qa.at[idxs[s4]], abufs[b],
                                  gsems[b]).wait()
            pltpu.make_async_copy(qb.at[idxd[s4]], bbufs[b],
                                  gsems[b]).wait()

        def compute(b):
            ab = abufs[b]
            bb = bbufs[b]
            def row(r, c2):
                for g in range(ngrp):
                    sl = pl.ds(g * 16, 16)
                    ab[r, sl] = jnp.maximum(ab[r, sl] + bb[r, sl], 0.0)
                return c2
            lax.fori_loop(0, KS, row, 0)

        def fire_s(s4, b):
            pltpu.async_copy(abufs[b], s_sh.at[idxd[s4]], sem_s, add=True)

        def drain_s(s4, b):
            pltpu.make_async_copy(abufs[b], s_sh.at[idxd[s4]],
                                  sem_s).wait()

        fire_idx(0, 0)
        fire_idx(1, 1)
        fire_idx(2, 2)
        fire_idx(3, 3)
        wait_idx(0, 0)
        fire_g(0, 0)
        wait_g(0, 0)
        compute(0)
        fire_s(0, 0)
        wait_idx(1, 1)
        fire_g(1, 1)

        NQ = (NCHS - 6) // 4
        def quad(j, carry):
            for c_off in (1, 2, 3, 4):
                c = 4 * j + c_off
                b = c_off % 2
                drain_s((c_off - 1) % 4, 1 - b)
                fire_idx(c + 3, (c_off + 3) % 4)
                wait_idx(c + 1, (c_off + 1) % 4)
                fire_g((c_off + 1) % 4, 1 - b)
                wait_g(c_off % 4, b)
                compute(b)
                fire_s(c_off % 4, b)
            return carry
        lax.fori_loop(0, NQ, quad, 0)

        for c in range(4 * NQ + 1, NCHS):
            b = c % 2
            drain_s((c - 1) % 4, 1 - b)
            if c + 3 <= NCHS - 1:
                fire_idx(c + 3, (c + 3) % 4)
            if c + 1 <= NCHS - 1:
                wait_idx(c + 1, (c + 1) % 4)
                fire_g((c + 1) % 4, 1 - b)
            wait_g(c % 4, b)
            compute(b)
            fire_s(c % 4, b)
        drain_s((NCHS - 1) % 4, (NCHS - 1) % 2)

        plsc.subcore_barrier()

        for j in range(RPT // KS):
            row0 = sid * RPT + j * KS
            pltpu.sync_copy(s_sh.at[pl.ds(row0, KS)], ab0)
            pltpu.sync_copy(ab0, out.at[cid, pl.ds(row0, KS)])

    return kern


_sc_scatter = _make_sc_scatter(False)


@functools.partial(
    pl.kernel,
    out_type=jax.ShapeDtypeStruct((NW, NP), jnp.float32),
    mesh=_MESH,
    compiler_params=pltpu.CompilerParams(use_tc_tiling_on_sc=False,
                                         needs_layout_passes=False),
    scratch_types=[
        pltpu.VMEM((K,), jnp.int32),
        pltpu.VMEM((K,), jnp.int32),
        pltpu.VMEM((K,), jnp.int32),
        pltpu.VMEM((K,), jnp.int32),
        pltpu.VMEM((NP,), jnp.float32),
        pltpu.SemaphoreType.DMA,
    ],
)
def _sc_deg(dst2, out, ix0, ix1, ix2, ix3, hist, sem_i):
    cid = lax.axis_index("c")
    sid = lax.axis_index("s")
    wid = sid * NC + cid
    idxs = (ix0, ix1, ix2, ix3)
    zero16 = jnp.zeros((16,), jnp.float32)
    ones16 = jnp.ones((16,), jnp.float32)

    def hzero(r, carry):
        hist[pl.ds(r * 16, 16)] = zero16
        return carry
    lax.fori_loop(0, NP // 16, hzero, 0)

    def fire(i, s4):
        pltpu.async_copy(dst2.at[wid * NCHUNK + i], idxs[s4], sem_i)

    def wait(i, s4):
        pltpu.make_async_copy(dst2.at[wid * NCHUNK + i], idxs[s4],
                              sem_i).wait()

    def process(s4):
        for g in range(K // 16):
            v = idxs[s4][pl.ds(g * 16, 16)]
            plsc.addupdate_scatter(hist, [v], ones16)

    fire(0, 0)
    fire(1, 1)
    fire(2, 2)
    fire(3, 3)

    def quad(j, carry):
        for c_off in (0, 1, 2, 3):
            c = 4 * j + c_off
            wait(c, c_off)
            process(c_off)
            fire(c + 4, c_off)
        return carry
    lax.fori_loop(0, (NCHUNK - 5) // 4, quad, 0)

    for c in range(4 * ((NCHUNK - 5) // 4), NCHUNK):
        s4 = c % 4
        wait(c, s4)
        process(s4)
        if c + 4 <= NCHUNK - 1:
            fire(c + 4, s4)

    pltpu.sync_copy(hist, out.at[wid])


@functools.partial(
    pl.kernel,
    out_type=jax.ShapeDtypeStruct((E, 2 * D), jnp.bfloat16),
    mesh=_MESH,
    compiler_params=pltpu.CompilerParams(use_tc_tiling_on_sc=False),
    scratch_types=[
        pltpu.VMEM((NCHUNK, K), jnp.int32),
        pltpu.VMEM((NCHUNK, K), jnp.int32),
        pltpu.VMEM((2, K, 2 * D), jnp.bfloat16),
        pltpu.VMEM((2, K, 2 * D), jnp.bfloat16),
        pltpu.VMEM((K, 2 * D), jnp.bfloat16),
        pltpu.SemaphoreType.DMA,
        pltpu.SemaphoreType.DMA,
        pltpu.SemaphoreType.DMA,
    ],
)
def _sc_edge(tt, src2, dst2, rd_out, idx2_s, idx2_d, gs, gd, gout,
             sem_g0, sem_g1, sem_st):
    cid = lax.axis_index("c")
    sid = lax.axis_index("s")
    wid = sid * NC + cid
    ngrp = D // 16
    sems = (sem_g0, sem_g1)

    pltpu.sync_copy(src2.at[pl.ds(wid * NCHUNK, NCHUNK)], idx2_s)
    pltpu.sync_copy(dst2.at[pl.ds(wid * NCHUNK, NCHUNK)], idx2_d)

    def fire(i, b):
        cp1 = pltpu.async_copy(tt.at[idx2_s.at[i]], gs.at[b], sems[b])
        cp2 = pltpu.async_copy(tt.at[idx2_d.at[i]], gd.at[b], sems[b])
        return cp1, cp2

    def wait_g(i, b):
        pltpu.make_async_copy(tt.at[idx2_s.at[i]], gs.at[b], sems[b]).wait()
        pltpu.make_async_copy(tt.at[idx2_d.at[i]], gd.at[b], sems[b]).wait()

    def compute(b):
        gsb = gs.at[b]
        gdb = gd.at[b]
        zero = jnp.zeros((32,), jnp.bfloat16)
        half = jnp.full((32,), 0.5, jnp.bfloat16)

        def row(r, c2):
            for g in range(D // 32):
                lo = pl.ds(g * 32, 32)
                hi = pl.ds(D + g * 32, 32)
                ru = jnp.maximum(gsb[r, lo] + gdb[r, hi], zero)
                rv = jnp.maximum(gdb[r, lo] + gsb[r, hi], zero)
                gout[r, lo] = (ru + rv) * half
                gout[r, hi] = ru - rv
            return c2
        lax.fori_loop(0, K, row, 0)

    def fire_store(i):
        return pltpu.async_copy(
            gout, rd_out.at[pl.ds(wid * EPW + i * K, K)], sem_st)

    def drain_store(i):
        pltpu.make_async_copy(
            gout, rd_out.at[pl.ds(wid * EPW + i * K, K)], sem_st).wait()

    fire(0, 0)
    fire(1, 1)
    wait_g(0, 0)
    compute(0)
    fire_store(0)
    fire(2, 0)

    def pair(j, carry):
        for (c_off, b) in ((1, 1), (2, 0)):
            c = 2 * j + c_off
            drain_store(c - 1)
            wait_g(c, b)
            compute(b)
            fire_store(c)
            fire(c + 2, b)
        return carry
    lax.fori_loop(0, 60, pair, 0)

    for c in (121, 122, 123, 124):
        b = c % 2
        drain_store(c - 1)
        wait_g(c, b)
        compute(b)
        fire_store(c)
        if c + 2 <= NCHUNK - 1:
            fire(c + 2, b)
    drain_store(NCHUNK - 1)


BN = 2000
BM = 1280
BE = 3200


def _dot(a, b):
    return jax.lax.dot_general(a, b, (((1,), (0,)), ((), ())),
                               preferred_element_type=jnp.float32)


def _prep_body(x_ref, w_ref, b_ref, qa_ref, qb_ref):
    x = x_ref[...]
    qa_ref[...] = _dot(x, w_ref[:D])
    qb_ref[...] = _dot(x, w_ref[D:]) + b_ref[...]


def _prep(x, w1, b1):
    return pl.pallas_call(
        _prep_body,
        grid=(N // BN,),
        in_specs=[
            pl.BlockSpec((BN, D), lambda i: (i, 0)),
            pl.BlockSpec((2 * D, D), lambda i: (0, 0)),
            pl.BlockSpec((1, D), lambda i: (0, 0)),
        ],
        out_specs=[pl.BlockSpec((BN, D), lambda i: (i, 0))] * 2,
        out_shape=[jax.ShapeDtypeStruct((N, D), jnp.float32)] * 2,
    )(x, w1, b1)


def _mid1_body(sp_ref, dp_ref, w2_ref, b2_ref, ew1_ref, eb1_ref, nw1_ref,
               nb1_ref, x1_ref, t1_ref, q2a_ref, q2b_ref):
    s = sp_ref[0] + sp_ref[1]
    deg = jax.lax.dot_general(dp_ref[...], jnp.ones((NW, 1), jnp.float32),
                              (((0,), (0,)), ((), ())),
                              preferred_element_type=jnp.float32)
    agg = _dot(s, w2_ref[...]) + deg * b2_ref[...]
    x1 = jnp.maximum(agg / jnp.maximum(deg, 1.0), 0.0)
    x1_ref[...] = x1
    t1_ref[:, :D] = _dot(x1, ew1_ref[:D]).astype(jnp.bfloat16)
    t1_ref[:, D:] = (_dot(x1, ew1_ref[D:]) + eb1_ref[...]).astype(jnp.bfloat16)
    mu = jnp.mean(x1, axis=1, keepdims=True)
    var = jnp.mean((x1 - mu) ** 2, axis=1, keepdims=True)
    xn = (x1 - mu) / (jnp.sqrt(var) + 1e-6)
    q2a_ref[...] = _dot(xn, nw1_ref[:D])
    q2b_ref[...] = _dot(xn, nw1_ref[D:]) + nb1_ref[...]


def _mid1(s1, degp, w2, b2, ew1, eb1, nw1, nb1):
    return pl.pallas_call(
        _mid1_body,
        grid=(NP // BM,),
        in_specs=[
            pl.BlockSpec((NC, BM, D), lambda i: (0, i, 0)),
            pl.BlockSpec((NW, BM), lambda i: (0, i)),
            pl.BlockSpec((D, D), lambda i: (0, 0)),
            pl.BlockSpec((1, D), lambda i: (0, 0)),
            pl.BlockSpec((2 * D, D), lambda i: (0, 0)),
            pl.BlockSpec((1, D), lambda i: (0, 0)),
            pl.BlockSpec((2 * D, D), lambda i: (0, 0)),
            pl.BlockSpec((1, D), lambda i: (0, 0)),
        ],
        out_specs=[
            pl.BlockSpec((BM, D), lambda i: (i, 0)),
            pl.BlockSpec((BM, 2 * D), lambda i: (i, 0)),
            pl.BlockSpec((BM, D), lambda i: (i, 0)),
            pl.BlockSpec((BM, D), lambda i: (i, 0)),
        ],
        out_shape=[
            jax.ShapeDtypeStruct((NP, D), jnp.float32),
            jax.ShapeDtypeStruct((NP, 2 * D), jnp.bfloat16),
            jax.ShapeDtypeStruct((NP, D), jnp.float32),
            jax.ShapeDtypeStruct((NP, D), jnp.float32),
        ],
    )(s1, degp, w2, b2, ew1, eb1, nw1, nb1)


def _mid2_body(dp_ref, s2p_ref, x1_ref, w2_ref, b2_ref, ew1_ref, eb1_ref,
               t2_ref):
    deg = jax.lax.dot_general(dp_ref[...], jnp.ones((NW, 1), jnp.float32),
                              (((0,), (0,)), ((), ())),
                              preferred_element_type=jnp.float32)
    s2 = s2p_ref[0] + s2p_ref[1]
    agg = _dot(s2, w2_ref[...]) + deg * b2_ref[...]
    x2 = x1_ref[...] + jnp.maximum(agg / jnp.maximum(deg, 1.0), 0.0)
    t2_ref[:, :D] = _dot(x2, ew1_ref[:D]).astype(jnp.bfloat16)
    t2_ref[:, D:] = (_dot(x2, ew1_ref[D:]) + eb1_ref[...]).astype(jnp.bfloat16)


def _mid2(degp, s2, x1, w2, b2, ew1, eb1):
    return pl.pallas_call(
        _mid2_body,
        grid=(NP // BM,),
        in_specs=[
            pl.BlockSpec((NW, BM), lambda i: (0, i)),
            pl.BlockSpec((NC, BM, D), lambda i: (0, i, 0)),
            pl.BlockSpec((BM, D), lambda i: (i, 0)),
            pl.BlockSpec((D, D), lambda i: (0, 0)),
            pl.BlockSpec((1, D), lambda i: (0, 0)),
            pl.BlockSpec((2 * D, D), lambda i: (0, 0)),
            pl.BlockSpec((1, D), lambda i: (0, 0)),
        ],
        out_specs=[pl.BlockSpec((BM, 2 * D), lambda i: (i, 0))],
        out_shape=[jax.ShapeDtypeStruct((NP, 2 * D), jnp.bfloat16)],
    )(degp, s2, x1, w2, b2, ew1, eb1)[0]


def _final_body(r1_ref, d1_ref, r2_ref, d2_ref, ang_ref,
                e1w2_ref, e1w3_ref, e1b2_ref, e1b3_ref,
                e2w2_ref, e2w3_ref, e2b2_ref, e2b3_ref,
                e2_ref, ssq_ref):
    i = pl.program_id(0)
    w3a1 = e1w3_ref[:D]
    v1 = e1w3_ref[D:D + 1]
    m1 = _dot(e1w2_ref[...], w3a1)
    c1 = _dot(e1b2_ref[...], w3a1) + e1b3_ref[...]
    w3a2 = e2w3_ref[:D]
    w3b2 = e2w3_ref[D:]
    a2 = _dot(e2w2_ref[...], w3a2)
    a1 = _dot(m1, w3b2)
    wv = _dot(v1, w3b2)
    cc = _dot(c1, w3b2) + _dot(e2b2_ref[...], w3a2) + e2b3_ref[...]

    e2 = (_dot(r2_ref[...], a2) + _dot(r1_ref[...], a1)
          + ang_ref[...] * wv + cc)
    e2_ref[...] = e2

    t1 = _dot(d1_ref[...], e1w2_ref[...])
    t2 = _dot(d2_ref[...], e2w2_ref[...])
    s1 = jnp.sum(t1 * t1)
    s2 = jnp.sum(t2 * t2)
    lane = lax.broadcasted_iota(jnp.int32, (1, 128), 1)
    contrib = jnp.where(lane == 0, s1, 0.0) + jnp.where(lane == 1, s2, 0.0)

    @pl.when(i == 0)
    def _():
        ssq_ref[...] = jnp.zeros_like(ssq_ref)
    ssq_ref[...] += contrib


def _final(r1, d1, r2, d2, ang, e1w2, e1w3, e1b2, e1b3, e2w2, e2w3, e2b2,
           e2b3):
    return pl.pallas_call(
        _final_body,
        grid=(E // BE,),
        in_specs=[
            pl.BlockSpec((BE, D), lambda i: (i, 0)),
            pl.BlockSpec((BE, D), lambda i: (i, 1)),
            pl.BlockSpec((BE, D), lambda i: (i, 0)),
            pl.BlockSpec((BE, D), lambda i: (i, 1)),
            pl.BlockSpec((BE, 1), lambda i: (i, 0)),
            pl.BlockSpec((D, D), lambda i: (0, 0)),
            pl.BlockSpec((D + 1, D), lambda i: (0, 0)),
            pl.BlockSpec((1, D), lambda i: (0, 0)),
            pl.BlockSpec((1, D), lambda i: (0, 0)),
            pl.BlockSpec((D, D), lambda i: (0, 0)),
            pl.BlockSpec((2 * D, OUT), lambda i: (0, 0)),
            pl.BlockSpec((1, D), lambda i: (0, 0)),
            pl.BlockSpec((1, OUT), lambda i: (0, 0)),
        ],
        out_specs=[
            pl.BlockSpec((BE, OUT), lambda i: (i, 0)),
            pl.BlockSpec((1, 128), lambda i: (0, 0)),
        ],
        out_shape=[
            jax.ShapeDtypeStruct((E, OUT), jnp.float32),
            jax.ShapeDtypeStruct((1, 128), jnp.float32),
        ],
    )(r1, d1, r2, d2, ang, e1w2, e1w3, e1b2, e1b3, e2w2, e2w3, e2b2, e2b3)


def kernel(node_features, edge_index, angles, gt_edges,
           nc1_W1, nc1_b1, nc1_W2, nc1_b2, nc2_W1, nc2_b1, nc2_W2, nc2_b2,
           ec1_W1, ec1_b1, ec1_W2, ec1_b2, ec1_W3, ec1_b3,
           ec2_W1, ec2_b1, ec2_W2, ec2_b2, ec2_W3, ec2_b3):
    src = edge_index[0]
    dst = edge_index[1]
    src2 = src.reshape(E // K, K)
    dst2 = dst.reshape(E // K, K)
    r2d = lambda b: b.reshape(1, -1)

    qa, qb = _prep(node_features, nc1_W1, r2d(nc1_b1))
    s1 = _sc_scatter(qa, qb, src2, dst2)[0]
    degp = _sc_deg(dst2)
    x1, t1, q2a, q2b = _mid1(s1, degp, nc1_W2, r2d(nc1_b2), ec1_W1,
                             r2d(ec1_b1), nc2_W1, r2d(nc2_b1))
    rd1 = _sc_edge(t1, src2, dst2)
    s2 = _sc_scatter(q2a, q2b, src2, dst2)[0]
    t2 = _mid2(degp, s2, x1, nc2_W2, r2d(nc2_b2), ec2_W1, r2d(ec2_b1))
    rd2 = _sc_edge(t2, src2, dst2)
    e2, ssq = _final(rd1, rd1, rd2, rd2, angles, ec1_W2, ec1_W3, r2d(ec1_b2),
                     r2d(ec1_b3), ec2_W2, ec2_W3, r2d(ec2_b2), r2d(ec2_b3))
    side = ((ssq[0, 0] + ssq[0, 1]) / (E * D) * 0.5).reshape(1)
    return e2, side
